# Initial kernel scaffold; baseline (speedup 1.0000x reference)
import jax
import jax.numpy as jnp
from jax.experimental import pallas as pl


def kernel(input, cloud_len_list, voxel_num, W1, W2, W3, W4, W5, g5, b5, Wl1, g6, b6, Wl2, bl2, g7, b7, Wl3, bl3, W6, gc6, bc6, W7, gc7, bc7, W8, gc8, bc8, Wl4, Wl5, bl5):
    raise NotImplementedError("write your pallas kernel here")


# speedup vs baseline: 2.1078x; 2.1078x over previous
import jax
import jax.numpy as jnp
from jax.experimental import pallas as pl

_PT = 512
_K = 16
_VK = 8
_VN = 32
_EMB = 1024
_NEG = -3.0e38
_DNT = (((1,), (1,)), ((), ()))
_DNN = (((1,), (0,)), ((), ()))


def _lr(t):
    return jnp.where(t >= 0, t, 0.2 * t)


def _dotb(a, b, dn=_DNT):
    return jax.lax.dot_general(a.astype(jnp.bfloat16), b.astype(jnp.bfloat16),
                               dn, preferred_element_type=jnp.float32)


def _gather_rows(onehot, tbl):
    return jax.lax.dot_general(onehot.astype(jnp.float32), tbl, _DNN,
                               preferred_element_type=jnp.float32,
                               precision=jax.lax.Precision.HIGHEST)


def _pairwise(xt):
    n = xt.shape[0]
    g = _dotb(xt, xt)
    xs = jnp.transpose(xt, (1, 0))
    xxc = jnp.sum(xs * xs, axis=0, keepdims=True)
    xx = jnp.transpose(xxc, (1, 0))
    pd = ((-xxc) - (-2.0 * g)) - xx
    sgn = jnp.where(xxc > 0, jnp.float32(1.0), jnp.float32(1e7))
    cols = jax.lax.broadcasted_iota(jnp.int32, (n, n), 1)
    return pd * sgn, cols


def _argmax_step(pdw, cols):
    n = pdw.shape[0]
    rmax = jnp.max(pdw, axis=1, keepdims=True)
    am = jnp.min(jnp.where(pdw == rmax, cols, n), axis=1, keepdims=True)
    onehot = cols == am
    return onehot, jnp.where(onehot, _NEG, pdw)


def _edge_max(xt, w, k):
    n, c = xt.shape
    o = w.shape[0]
    pd, cols = _pairwise(xt)

    def step(_, carry):
        pdw, m = carry
        onehot, pdw = _argmax_step(pdw, cols)
        gath = _gather_rows(onehot, xt)
        fm = jnp.concatenate([gath - xt, xt], axis=1)
        ym = _dotb(fm, w)
        return pdw, jnp.maximum(m, _lr(ym))

    init = (pd, jnp.full((n, o), _NEG, jnp.float32))
    _, m = jax.lax.fori_loop(0, k, step, init)
    return m


def _k1_body(x_ref, w1_ref, w2_ref, w3_ref, w4_ref, w5_ref, y_ref):
    xt = x_ref[0]
    x1 = _edge_max(xt, w1_ref[...], _K)
    x2 = _edge_max(x1, w2_ref[...], _K)
    x3 = _edge_max(x2, w3_ref[...], _K)
    x4 = _edge_max(x3, w4_ref[...], _K)
    xcat = jnp.concatenate([x1, x2, x3, x4], axis=1)
    y_ref[0] = jnp.transpose(_dotb(xcat, w5_ref[...]), (1, 0))


def _k2_body(y_ref, g_ref, b_ref, mx_ref, mn_ref):
    y = y_ref[...]
    ct = y.shape[1]
    tot = jnp.sum(jnp.sum(y, axis=2, keepdims=True), axis=0, keepdims=True)
    m = tot / (_VN * _PT)
    d = y - m
    vt = jnp.sum(jnp.sum(d * d, axis=2, keepdims=True), axis=0, keepdims=True)
    v = vt / (_VN * _PT)
    g3 = g_ref[...].reshape(1, ct, 1)
    b3 = b_ref[...].reshape(1, ct, 1)
    z = _lr(g3 * d / jnp.sqrt(v + 1e-5) + b3)
    mx_ref[...] = jnp.max(z, axis=2)
    mn_ref[...] = jnp.mean(z, axis=2)


def _bn_rows(x, g, b):
    m = jnp.mean(x, axis=0, keepdims=True)
    d = x - m
    v = jnp.mean(d * d, axis=0, keepdims=True)
    return g * d / jnp.sqrt(v + 1e-5) + b


def _vox_layer(xt, w, g, b, k):
    n, c = xt.shape
    o = w.shape[0]
    pd, cols = _pairwise(xt)
    ys = []
    pdw = pd
    for _ in range(k):
        onehot, pdw = _argmax_step(pdw, cols)
        gath = _gather_rows(onehot, xt)
        fm = jnp.concatenate([gath - xt, xt], axis=1)
        ys.append(_dotb(fm, w))
    tot = ys[0]
    for y in ys[1:]:
        tot = tot + y
    mean = jnp.sum(tot, axis=0, keepdims=True) / (n * k)
    vtot = jnp.zeros((1, o), jnp.float32)
    for y in ys:
        d = y - mean
        vtot = vtot + jnp.sum(d * d, axis=0, keepdims=True)
    denom = jnp.sqrt(vtot / (n * k) + 1e-5)
    m = jnp.full((n, o), _NEG, jnp.float32)
    for y in ys:
        m = jnp.maximum(m, _lr(g * (y - mean) / denom + b))
    return m


def _k3_body(h_ref, wl1_ref, g6_ref, b6_ref, wl2_ref, bl2_ref, g7_ref,
             b7_ref, wl3_ref, bl3_ref, w6_ref, gc6_ref, bc6_ref, w7_ref,
             gc7_ref, bc7_ref, w8_ref, gc8_ref, bc8_ref, wl4_ref, wl5_ref,
             bl5_ref, out_ref):
    h = h_ref[...]
    h = _lr(_bn_rows(_dotb(h, wl1_ref[...], _DNN), g6_ref[...], b6_ref[...]))
    h = _lr(_bn_rows(_dotb(h, wl2_ref[...], _DNN) + bl2_ref[...],
                     g7_ref[...], b7_ref[...]))
    xt = _dotb(h, wl3_ref[...], _DNN) + bl3_ref[...]
    x1 = _vox_layer(xt, w6_ref[...], gc6_ref[...], bc6_ref[...], _VK)
    x2 = _vox_layer(x1, w7_ref[...], gc7_ref[...], bc7_ref[...], _VK)
    xcat = jnp.concatenate([x1, x2], axis=1)
    z8 = _lr(_bn_rows(_dotb(xcat, w8_ref[...]), gc8_ref[...], bc8_ref[...]))
    pmax = jnp.max(z8, axis=0, keepdims=True)
    pmean = jnp.mean(z8, axis=0, keepdims=True)
    hv = jnp.concatenate([pmax, pmean], axis=1)
    hv = _lr(_dotb(hv, wl4_ref[...], _DNN))
    out_ref[...] = _dotb(hv, wl5_ref[...], _DNN) + bl5_ref[...]


def kernel(input, cloud_len_list, voxel_num, W1, W2, W3, W4, W5, g5, b5,
           Wl1, g6, b6, Wl2, bl2, g7, b7, Wl3, bl3, W6, gc6, bc6, W7, gc7,
           bc7, W8, gc8, bc8, Wl4, Wl5, bl5):
    x = input.reshape(_VN, _PT, 9)
    y5 = pl.pallas_call(
        _k1_body,
        grid=(_VN,),
        in_specs=[
            pl.BlockSpec((1, _PT, 9), lambda v: (v, 0, 0)),
            pl.BlockSpec(W1.shape, lambda v: (0, 0)),
            pl.BlockSpec(W2.shape, lambda v: (0, 0)),
            pl.BlockSpec(W3.shape, lambda v: (0, 0)),
            pl.BlockSpec(W4.shape, lambda v: (0, 0)),
            pl.BlockSpec(W5.shape, lambda v: (0, 0)),
        ],
        out_specs=pl.BlockSpec((1, _EMB, _PT), lambda v: (v, 0, 0)),
        out_shape=jax.ShapeDtypeStruct((_VN, _EMB, _PT), jnp.float32),
    )(x, W1, W2, W3, W4, W5)

    ct = 128
    pmax, pmean = pl.pallas_call(
        _k2_body,
        grid=(_EMB // ct,),
        in_specs=[
            pl.BlockSpec((_VN, ct, _PT), lambda c: (0, c, 0)),
            pl.BlockSpec((1, ct), lambda c: (0, c)),
            pl.BlockSpec((1, ct), lambda c: (0, c)),
        ],
        out_specs=[
            pl.BlockSpec((_VN, ct), lambda c: (0, c)),
            pl.BlockSpec((_VN, ct), lambda c: (0, c)),
        ],
        out_shape=[
            jax.ShapeDtypeStruct((_VN, _EMB), jnp.float32),
            jax.ShapeDtypeStruct((_VN, _EMB), jnp.float32),
        ],
    )(y5, g5.reshape(1, -1), b5.reshape(1, -1))

    pooled = jnp.concatenate([pmax, pmean], axis=1)
    out = pl.pallas_call(
        _k3_body,
        out_shape=jax.ShapeDtypeStruct((1, 40), jnp.float32),
    )(pooled, Wl1, g6.reshape(1, -1), b6.reshape(1, -1), Wl2,
      bl2.reshape(1, -1), g7.reshape(1, -1), b7.reshape(1, -1), Wl3,
      bl3.reshape(1, -1), W6, gc6.reshape(1, -1), bc6.reshape(1, -1), W7,
      gc7.reshape(1, -1), bc7.reshape(1, -1), W8, gc8.reshape(1, -1),
      bc8.reshape(1, -1), Wl4, Wl5, bl5.reshape(1, -1))
    return out


# speedup vs baseline: 2.3376x; 1.1090x over previous
import jax
import jax.numpy as jnp
from jax.experimental import pallas as pl

_PT = 512
_K = 16
_VK = 8
_VN = 32
_EMB = 1024
_NEG = -3.0e38
_DNT = (((1,), (1,)), ((), ()))
_DNN = (((1,), (0,)), ((), ()))


def _lr(t):
    return jnp.where(t >= 0, t, 0.2 * t)


def _dotb(a, b, dn=_DNT):
    return jax.lax.dot_general(a.astype(jnp.bfloat16), b.astype(jnp.bfloat16),
                               dn, preferred_element_type=jnp.float32)


def _split3(x):
    h1 = x.astype(jnp.bfloat16)
    r1 = x - h1.astype(jnp.float32)
    h2 = r1.astype(jnp.bfloat16)
    r2 = r1 - h2.astype(jnp.float32)
    h3 = r2.astype(jnp.bfloat16)
    return h1, h2, h3


def _gather3(oh, parts):
    h1, h2, h3 = parts
    g1 = jax.lax.dot_general(oh, h1, _DNN, preferred_element_type=jnp.float32)
    g2 = jax.lax.dot_general(oh, h2, _DNN, preferred_element_type=jnp.float32)
    g3 = jax.lax.dot_general(oh, h3, _DNN, preferred_element_type=jnp.float32)
    return (g1 + g2) + g3


def _pairwise(xt):
    n = xt.shape[0]
    g = _dotb(xt, xt)
    xs = jnp.transpose(xt, (1, 0))
    xxc = jnp.sum(xs * xs, axis=0, keepdims=True)
    xx = jnp.transpose(xxc, (1, 0))
    pd = ((-xxc) - (-2.0 * g)) - xx
    sgn = jnp.where(xxc > 0, jnp.float32(1.0), jnp.float32(1e7))
    cols = jax.lax.broadcasted_iota(jnp.int32, (n, n), 1)
    return pd * sgn, cols


def _argmax_step(pdw, cols):
    am = jnp.argmax(pdw, axis=1)
    onehot = cols == am[:, None]
    return onehot, jnp.where(onehot, _NEG, pdw)


def _edge_max(xt, w, k):
    n, c = xt.shape
    o = w.shape[0]
    pd, cols = _pairwise(xt)
    parts = _split3(xt)
    wb = w.astype(jnp.bfloat16)

    def step(_, carry):
        pdw, m = carry
        onehot, pdw = _argmax_step(pdw, cols)
        gath = _gather3(onehot.astype(jnp.bfloat16), parts)
        fm = jnp.concatenate([gath - xt, xt], axis=1)
        ym = jax.lax.dot_general(fm.astype(jnp.bfloat16), wb, _DNT,
                                 preferred_element_type=jnp.float32)
        return pdw, jnp.maximum(m, _lr(ym))

    init = (pd, jnp.full((n, o), _NEG, jnp.float32))
    _, m = jax.lax.fori_loop(0, k, step, init)
    return m


def _k1_body(x_ref, w1_ref, w2_ref, w3_ref, w4_ref, w5_ref, y_ref):
    xt = x_ref[0]
    x1 = _edge_max(xt, w1_ref[...], _K)
    x2 = _edge_max(x1, w2_ref[...], _K)
    x3 = _edge_max(x2, w3_ref[...], _K)
    x4 = _edge_max(x3, w4_ref[...], _K)
    xcat = jnp.concatenate([x1, x2, x3, x4], axis=1)
    y_ref[0] = jnp.transpose(_dotb(xcat, w5_ref[...]), (1, 0))


def _k2_body(y_ref, g_ref, b_ref, mx_ref, mn_ref):
    y = y_ref[...]
    ct = y.shape[1]
    tot = jnp.sum(jnp.sum(y, axis=2, keepdims=True), axis=0, keepdims=True)
    m = tot / (_VN * _PT)
    d = y - m
    vt = jnp.sum(jnp.sum(d * d, axis=2, keepdims=True), axis=0, keepdims=True)
    v = vt / (_VN * _PT)
    g3 = g_ref[...].reshape(1, ct, 1)
    b3 = b_ref[...].reshape(1, ct, 1)
    z = _lr(g3 * d / jnp.sqrt(v + 1e-5) + b3)
    mx_ref[...] = jnp.max(z, axis=2)
    mn_ref[...] = jnp.mean(z, axis=2)


def _bn_rows(x, g, b):
    m = jnp.mean(x, axis=0, keepdims=True)
    d = x - m
    v = jnp.mean(d * d, axis=0, keepdims=True)
    return g * d / jnp.sqrt(v + 1e-5) + b


def _vox_layer(xt, w, g, b, k):
    n, c = xt.shape
    o = w.shape[0]
    pd, cols = _pairwise(xt)
    parts = _split3(xt)
    ys = []
    pdw = pd
    for _ in range(k):
        onehot, pdw = _argmax_step(pdw, cols)
        gath = _gather3(onehot.astype(jnp.bfloat16), parts)
        fm = jnp.concatenate([gath - xt, xt], axis=1)
        ys.append(_dotb(fm, w))
    tot = ys[0]
    for y in ys[1:]:
        tot = tot + y
    mean = jnp.sum(tot, axis=0, keepdims=True) / (n * k)
    vtot = jnp.zeros((1, o), jnp.float32)
    for y in ys:
        d = y - mean
        vtot = vtot + jnp.sum(d * d, axis=0, keepdims=True)
    denom = jnp.sqrt(vtot / (n * k) + 1e-5)
    m = jnp.full((n, o), _NEG, jnp.float32)
    for y in ys:
        m = jnp.maximum(m, _lr(g * (y - mean) / denom + b))
    return m


def _k3_body(h_ref, wl1_ref, g6_ref, b6_ref, wl2_ref, bl2_ref, g7_ref,
             b7_ref, wl3_ref, bl3_ref, w6_ref, gc6_ref, bc6_ref, w7_ref,
             gc7_ref, bc7_ref, w8_ref, gc8_ref, bc8_ref, wl4_ref, wl5_ref,
             bl5_ref, out_ref):
    h = h_ref[...]
    h = _lr(_bn_rows(_dotb(h, wl1_ref[...], _DNN), g6_ref[...], b6_ref[...]))
    h = _lr(_bn_rows(_dotb(h, wl2_ref[...], _DNN) + bl2_ref[...],
                     g7_ref[...], b7_ref[...]))
    xt = _dotb(h, wl3_ref[...], _DNN) + bl3_ref[...]
    x1 = _vox_layer(xt, w6_ref[...], gc6_ref[...], bc6_ref[...], _VK)
    x2 = _vox_layer(x1, w7_ref[...], gc7_ref[...], bc7_ref[...], _VK)
    xcat = jnp.concatenate([x1, x2], axis=1)
    z8 = _lr(_bn_rows(_dotb(xcat, w8_ref[...]), gc8_ref[...], bc8_ref[...]))
    pmax = jnp.max(z8, axis=0, keepdims=True)
    pmean = jnp.mean(z8, axis=0, keepdims=True)
    hv = jnp.concatenate([pmax, pmean], axis=1)
    hv = _lr(_dotb(hv, wl4_ref[...], _DNN))
    out_ref[...] = _dotb(hv, wl5_ref[...], _DNN) + bl5_ref[...]


def kernel(input, cloud_len_list, voxel_num, W1, W2, W3, W4, W5, g5, b5,
           Wl1, g6, b6, Wl2, bl2, g7, b7, Wl3, bl3, W6, gc6, bc6, W7, gc7,
           bc7, W8, gc8, bc8, Wl4, Wl5, bl5):
    x = input.reshape(_VN, _PT, 9)
    y5 = pl.pallas_call(
        _k1_body,
        grid=(_VN,),
        in_specs=[
            pl.BlockSpec((1, _PT, 9), lambda v: (v, 0, 0)),
            pl.BlockSpec(W1.shape, lambda v: (0, 0)),
            pl.BlockSpec(W2.shape, lambda v: (0, 0)),
            pl.BlockSpec(W3.shape, lambda v: (0, 0)),
            pl.BlockSpec(W4.shape, lambda v: (0, 0)),
            pl.BlockSpec(W5.shape, lambda v: (0, 0)),
        ],
        out_specs=pl.BlockSpec((1, _EMB, _PT), lambda v: (v, 0, 0)),
        out_shape=jax.ShapeDtypeStruct((_VN, _EMB, _PT), jnp.float32),
    )(x, W1, W2, W3, W4, W5)

    ct = 128
    pmax, pmean = pl.pallas_call(
        _k2_body,
        grid=(_EMB // ct,),
        in_specs=[
            pl.BlockSpec((_VN, ct, _PT), lambda c: (0, c, 0)),
            pl.BlockSpec((1, ct), lambda c: (0, c)),
            pl.BlockSpec((1, ct), lambda c: (0, c)),
        ],
        out_specs=[
            pl.BlockSpec((_VN, ct), lambda c: (0, c)),
            pl.BlockSpec((_VN, ct), lambda c: (0, c)),
        ],
        out_shape=[
            jax.ShapeDtypeStruct((_VN, _EMB), jnp.float32),
            jax.ShapeDtypeStruct((_VN, _EMB), jnp.float32),
        ],
    )(y5, g5.reshape(1, -1), b5.reshape(1, -1))

    pooled = jnp.concatenate([pmax, pmean], axis=1)
    out = pl.pallas_call(
        _k3_body,
        out_shape=jax.ShapeDtypeStruct((1, 40), jnp.float32),
    )(pooled, Wl1, g6.reshape(1, -1), b6.reshape(1, -1), Wl2,
      bl2.reshape(1, -1), g7.reshape(1, -1), b7.reshape(1, -1), Wl3,
      bl3.reshape(1, -1), W6, gc6.reshape(1, -1), bc6.reshape(1, -1), W7,
      gc7.reshape(1, -1), bc7.reshape(1, -1), W8, gc8.reshape(1, -1),
      bc8.reshape(1, -1), Wl4, Wl5, bl5.reshape(1, -1))
    return out


# speedup vs baseline: 3.5918x; 1.5365x over previous
import jax
import jax.numpy as jnp
from jax.experimental import pallas as pl

_PT = 512
_K = 16
_VK = 8
_VN = 32
_EMB = 1024
_NEG = -3.0e38
_DNT = (((1,), (1,)), ((), ()))
_DNN = (((1,), (0,)), ((), ()))


def _lr(t):
    return jnp.where(t >= 0, t, 0.2 * t)


def _dotb(a, b, dn=_DNT):
    return jax.lax.dot_general(a.astype(jnp.bfloat16), b.astype(jnp.bfloat16),
                               dn, preferred_element_type=jnp.float32)


def _split3(x):
    h1 = x.astype(jnp.bfloat16)
    r1 = x - h1.astype(jnp.float32)
    h2 = r1.astype(jnp.bfloat16)
    r2 = r1 - h2.astype(jnp.float32)
    h3 = r2.astype(jnp.bfloat16)
    return h1, h2, h3


def _gather3(oh, tbl3, c):
    y = jax.lax.dot_general(oh, tbl3, _DNN, preferred_element_type=jnp.float32)
    return (y[:, :c] + y[:, c:2 * c]) + y[:, 2 * c:3 * c]


def _pairwise(xt):
    n = xt.shape[0]
    g = _dotb(xt, xt)
    xs = jnp.transpose(xt, (1, 0))
    xxc = jnp.sum(xs * xs, axis=0, keepdims=True)
    xx = jnp.transpose(xxc, (1, 0))
    pd = ((-xxc) - (-2.0 * g)) - xx
    sgn = jnp.where(xxc > 0, jnp.float32(1.0), jnp.float32(1e7))
    cols = jax.lax.broadcasted_iota(jnp.int32, (n, n), 1)
    return pd * sgn, cols


def _argmax_step(pdw, cols):
    am = jnp.argmax(pdw, axis=1)
    onehot = cols == am[:, None]
    return onehot, jnp.where(onehot, _NEG, pdw)


def _edge_max2(xa, xb, w, k):
    n, c = xa.shape
    o = w.shape[0]
    pda, cols = _pairwise(xa)
    pdb, _ = _pairwise(xb)
    ta = jnp.concatenate(_split3(xa), axis=1)
    tb = jnp.concatenate(_split3(xb), axis=1)
    wb = w.astype(jnp.bfloat16)

    def conv(oh, tbl3, xt):
        gath = _gather3(oh.astype(jnp.bfloat16), tbl3, c)
        fm = jnp.concatenate([gath - xt, xt], axis=1)
        return jax.lax.dot_general(fm.astype(jnp.bfloat16), wb, _DNT,
                                   preferred_element_type=jnp.float32)

    def step(_, carry):
        pa, ma, pb, mb = carry
        oha, pa = _argmax_step(pa, cols)
        ohb, pb = _argmax_step(pb, cols)
        ma = jnp.maximum(ma, _lr(conv(oha, ta, xa)))
        mb = jnp.maximum(mb, _lr(conv(ohb, tb, xb)))
        return pa, ma, pb, mb

    init = (pda, jnp.full((n, o), _NEG, jnp.float32),
            pdb, jnp.full((n, o), _NEG, jnp.float32))
    _, ma, _, mb = jax.lax.fori_loop(0, k, step, init)
    return ma, mb


def _k1_body(x_ref, w1_ref, w2_ref, w3_ref, w4_ref, w5_ref, y_ref):
    xa, xb = x_ref[0], x_ref[1]
    x1a, x1b = _edge_max2(xa, xb, w1_ref[...], _K)
    x2a, x2b = _edge_max2(x1a, x1b, w2_ref[...], _K)
    x3a, x3b = _edge_max2(x2a, x2b, w3_ref[...], _K)
    x4a, x4b = _edge_max2(x3a, x3b, w4_ref[...], _K)
    xca = jnp.concatenate([x1a, x2a, x3a, x4a], axis=1)
    xcb = jnp.concatenate([x1b, x2b, x3b, x4b], axis=1)
    w5 = w5_ref[...]
    y_ref[0] = jnp.transpose(_dotb(xca, w5), (1, 0))
    y_ref[1] = jnp.transpose(_dotb(xcb, w5), (1, 0))


def _k2_body(y_ref, g_ref, b_ref, mx_ref, mn_ref):
    y = y_ref[...]
    ct = y.shape[1]
    tot = jnp.sum(jnp.sum(y, axis=2, keepdims=True), axis=0, keepdims=True)
    m = tot / (_VN * _PT)
    d = y - m
    vt = jnp.sum(jnp.sum(d * d, axis=2, keepdims=True), axis=0, keepdims=True)
    v = vt / (_VN * _PT)
    g3 = g_ref[...].reshape(1, ct, 1)
    b3 = b_ref[...].reshape(1, ct, 1)
    z = _lr(g3 * d / jnp.sqrt(v + 1e-5) + b3)
    mx_ref[...] = jnp.max(z, axis=2)
    mn_ref[...] = jnp.mean(z, axis=2)


def _bn_rows(x, g, b):
    m = jnp.mean(x, axis=0, keepdims=True)
    d = x - m
    v = jnp.mean(d * d, axis=0, keepdims=True)
    return g * d / jnp.sqrt(v + 1e-5) + b


def _vox_layer(xt, w, g, b, k):
    n, c = xt.shape
    o = w.shape[0]
    pd, cols = _pairwise(xt)
    parts = jnp.concatenate(_split3(xt), axis=1)
    ys = []
    pdw = pd
    for _ in range(k):
        onehot, pdw = _argmax_step(pdw, cols)
        gath = _gather3(onehot.astype(jnp.bfloat16), parts, c)
        fm = jnp.concatenate([gath - xt, xt], axis=1)
        ys.append(_dotb(fm, w))
    tot = ys[0]
    for y in ys[1:]:
        tot = tot + y
    mean = jnp.sum(tot, axis=0, keepdims=True) / (n * k)
    vtot = jnp.zeros((1, o), jnp.float32)
    for y in ys:
        d = y - mean
        vtot = vtot + jnp.sum(d * d, axis=0, keepdims=True)
    denom = jnp.sqrt(vtot / (n * k) + 1e-5)
    m = jnp.full((n, o), _NEG, jnp.float32)
    for y in ys:
        m = jnp.maximum(m, _lr(g * (y - mean) / denom + b))
    return m


def _k3_body(h_ref, wl1_ref, g6_ref, b6_ref, wl2_ref, bl2_ref, g7_ref,
             b7_ref, wl3_ref, bl3_ref, w6_ref, gc6_ref, bc6_ref, w7_ref,
             gc7_ref, bc7_ref, w8_ref, gc8_ref, bc8_ref, wl4_ref, wl5_ref,
             bl5_ref, out_ref):
    h = h_ref[...]
    h = _lr(_bn_rows(_dotb(h, wl1_ref[...], _DNN), g6_ref[...], b6_ref[...]))
    h = _lr(_bn_rows(_dotb(h, wl2_ref[...], _DNN) + bl2_ref[...],
                     g7_ref[...], b7_ref[...]))
    xt = _dotb(h, wl3_ref[...], _DNN) + bl3_ref[...]
    x1 = _vox_layer(xt, w6_ref[...], gc6_ref[...], bc6_ref[...], _VK)
    x2 = _vox_layer(x1, w7_ref[...], gc7_ref[...], bc7_ref[...], _VK)
    xcat = jnp.concatenate([x1, x2], axis=1)
    z8 = _lr(_bn_rows(_dotb(xcat, w8_ref[...]), gc8_ref[...], bc8_ref[...]))
    pmax = jnp.max(z8, axis=0, keepdims=True)
    pmean = jnp.mean(z8, axis=0, keepdims=True)
    hv = jnp.concatenate([pmax, pmean], axis=1)
    hv = _lr(_dotb(hv, wl4_ref[...], _DNN))
    out_ref[...] = _dotb(hv, wl5_ref[...], _DNN) + bl5_ref[...]


def kernel(input, cloud_len_list, voxel_num, W1, W2, W3, W4, W5, g5, b5,
           Wl1, g6, b6, Wl2, bl2, g7, b7, Wl3, bl3, W6, gc6, bc6, W7, gc7,
           bc7, W8, gc8, bc8, Wl4, Wl5, bl5):
    x = input.reshape(_VN, _PT, 9)
    y5 = pl.pallas_call(
        _k1_body,
        grid=(_VN // 2,),
        in_specs=[
            pl.BlockSpec((2, _PT, 9), lambda v: (v, 0, 0)),
            pl.BlockSpec(W1.shape, lambda v: (0, 0)),
            pl.BlockSpec(W2.shape, lambda v: (0, 0)),
            pl.BlockSpec(W3.shape, lambda v: (0, 0)),
            pl.BlockSpec(W4.shape, lambda v: (0, 0)),
            pl.BlockSpec(W5.shape, lambda v: (0, 0)),
        ],
        out_specs=pl.BlockSpec((2, _EMB, _PT), lambda v: (v, 0, 0)),
        out_shape=jax.ShapeDtypeStruct((_VN, _EMB, _PT), jnp.float32),
    )(x, W1, W2, W3, W4, W5)

    ct = 128
    pmax, pmean = pl.pallas_call(
        _k2_body,
        grid=(_EMB // ct,),
        in_specs=[
            pl.BlockSpec((_VN, ct, _PT), lambda c: (0, c, 0)),
            pl.BlockSpec((1, ct), lambda c: (0, c)),
            pl.BlockSpec((1, ct), lambda c: (0, c)),
        ],
        out_specs=[
            pl.BlockSpec((_VN, ct), lambda c: (0, c)),
            pl.BlockSpec((_VN, ct), lambda c: (0, c)),
        ],
        out_shape=[
            jax.ShapeDtypeStruct((_VN, _EMB), jnp.float32),
            jax.ShapeDtypeStruct((_VN, _EMB), jnp.float32),
        ],
    )(y5, g5.reshape(1, -1), b5.reshape(1, -1))

    pooled = jnp.concatenate([pmax, pmean], axis=1)
    out = pl.pallas_call(
        _k3_body,
        out_shape=jax.ShapeDtypeStruct((1, 40), jnp.float32),
    )(pooled, Wl1, g6.reshape(1, -1), b6.reshape(1, -1), Wl2,
      bl2.reshape(1, -1), g7.reshape(1, -1), b7.reshape(1, -1), Wl3,
      bl3.reshape(1, -1), W6, gc6.reshape(1, -1), bc6.reshape(1, -1), W7,
      gc7.reshape(1, -1), bc7.reshape(1, -1), W8, gc8.reshape(1, -1),
      bc8.reshape(1, -1), Wl4, Wl5, bl5.reshape(1, -1))
    return out
